# baseline (speedup 1.0000x reference)
import functools

import jax
import jax.numpy as jnp
from jax.experimental import pallas as pl
from jax.experimental.pallas import tpu as pltpu
from jax.experimental.pallas import tpu_sc as plsc

_S, _D = 2048, 1024
_H, _HKV, _HD = 16, 4, 64
_E, _DFF = 8, 512
_G = _H // _HKV
_QKVD = (_H + 2 * _HKV) * _HD
_EPS = 1e-6
_THETA = 10000.0
_ST = 256
_QT = 512
_T = 128
_NT = (_S * 2) // _T + _E
_NTT = _NT * _T
_NW = 32
_CH = _S // _NW


def _stage1_kernel(pos_ref, inv_ref, x_ref, w1_ref, wqkv_ref,
                   qkv_ref, cos_ref, sin_ref):
    x = x_ref[...]
    v = jnp.mean(x * x, axis=-1, keepdims=True)
    xn = w1_ref[...] * (x * jax.lax.rsqrt(v + _EPS))
    qkv_ref[...] = jax.lax.dot_general(
        xn.astype(jnp.bfloat16), wqkv_ref[...],
        (((1,), (1,)), ((), ())), preferred_element_type=jnp.float32)
    f = pos_ref[...] * inv_ref[...]
    cos_ref[...] = jnp.cos(f)
    sin_ref[...] = jnp.sin(f)


def _rope(xn, c, s):
    x1 = xn[:, : _HD // 2]
    x2 = xn[:, _HD // 2:]
    return jnp.concatenate([x1 * c - x2 * s, x2 * c + x1 * s], axis=-1)


def _attn_kernel(cos_ref, sin_ref, qln_ref, kln_ref, q_ref, kv_ref,
                 ctx_ref):
    t = pl.program_id(0)
    h2 = pl.program_id(1)
    g2 = h2 // 2
    cf = cos_ref[...]
    sf = sin_ref[...]
    cq = cos_ref[pl.ds(t * _QT, _QT), :]
    sq = sin_ref[pl.ds(t * _QT, _QT), :]

    kv = kv_ref[...]
    ksel = jnp.where(
        g2 < 2,
        jnp.where(g2 == 0, kv[:, 0:_HD], kv[:, _HD:2 * _HD]),
        jnp.where(g2 == 2, kv[:, 2 * _HD:3 * _HD], kv[:, 3 * _HD:4 * _HD]))
    vsel = jnp.where(
        g2 < 2,
        jnp.where(g2 == 0, kv[:, 4 * _HD:5 * _HD], kv[:, 5 * _HD:6 * _HD]),
        jnp.where(g2 == 2, kv[:, 6 * _HD:7 * _HD], kv[:, 7 * _HD:8 * _HD]))
    kvar = jnp.mean(ksel * ksel, axis=-1, keepdims=True)
    kn = kln_ref[...] * (ksel * jax.lax.rsqrt(kvar + _EPS))
    kr = _rope(kn, cf, sf).astype(jnp.bfloat16)
    vb = vsel.astype(jnp.bfloat16)

    qpair = q_ref[...]

    def head(q):
        qv = jnp.mean(q * q, axis=-1, keepdims=True)
        qn = qln_ref[...] * (q * jax.lax.rsqrt(qv + _EPS))
        qr = _rope(qn, cq, sq).astype(jnp.bfloat16)
        scores = jax.lax.dot_general(
            qr, kr, (((1,), (1,)), ((), ())),
            preferred_element_type=jnp.float32) * (1.0 / 8.0)
        m = jnp.max(scores, axis=-1, keepdims=True)
        p = jnp.exp(scores - m)
        attn = (p / jnp.sum(p, axis=-1, keepdims=True)).astype(jnp.bfloat16)
        return jax.lax.dot_general(
            attn, vb, (((1,), (0,)), ((), ())),
            preferred_element_type=jnp.float32)

    ctx_ref[...] = jnp.concatenate(
        [head(qpair[:, :_HD]), head(qpair[:, _HD:])], axis=-1)


def _stage3_kernel(res_ref, ctx_ref, wo_ref, ln2_ref, wr_ref,
                   h_ref, x2b_ref, we_ref, cnt_ref):
    i = pl.program_id(0)
    ao = jax.lax.dot_general(
        ctx_ref[...].astype(jnp.bfloat16), wo_ref[...],
        (((1,), (1,)), ((), ())), preferred_element_type=jnp.float32)
    h = res_ref[...] + ao
    h_ref[...] = h
    v = jnp.mean(h * h, axis=-1, keepdims=True)
    x2 = ln2_ref[...] * (h * jax.lax.rsqrt(v + _EPS))
    x2b_ref[...] = x2.astype(jnp.bfloat16)
    logits = jax.lax.dot_general(
        x2, wr_ref[...], (((1,), (1,)), ((), ())),
        preferred_element_type=jnp.float32)
    lm = jnp.max(logits, axis=-1, keepdims=True)
    el = jnp.exp(logits - lm)
    p = el / jnp.sum(el, axis=-1, keepdims=True)
    it = jax.lax.broadcasted_iota(jnp.int32, p.shape, 1)
    m1 = jnp.max(p, axis=-1, keepdims=True)
    i1 = jnp.min(jnp.where(p == m1, it, _E), axis=-1, keepdims=True)
    p2 = jnp.where(it == i1, -1.0, p)
    m2 = jnp.max(p2, axis=-1, keepdims=True)
    i2 = jnp.min(jnp.where(p2 == m2, it, _E), axis=-1, keepdims=True)
    we = jnp.where(it == i1, m1, 0.0) + jnp.where(it == i2, m2, 0.0)
    we = we / (m1 + m2)
    we_ref[...] = we
    csum = jnp.sum((we > 0.0).astype(jnp.float32), axis=0, keepdims=True)

    @pl.when(i == 0)
    def _():
        cnt_ref[...] = jnp.zeros_like(cnt_ref)

    cnt_ref[...] += csum


def _meta_kernel(cnt_ref, we_ref, dlo_ref, dhi_ref, wlo_ref, whi_ref,
                 te_ref, base_ref, carry_ref):
    i = pl.program_id(0)

    @pl.when(i == 0)
    def _():
        cnt = cnt_ref[...]
        pad = jnp.ceil(cnt * (1.0 / _T)) * float(_T)
        eu = jax.lax.broadcasted_iota(jnp.int32, (_E, _E), 0)
        ev = jax.lax.broadcasted_iota(jnp.int32, (_E, _E), 1)
        tri = (eu < ev).astype(jnp.bfloat16)
        base = jax.lax.dot_general(
            pad.astype(jnp.bfloat16), tri, (((1,), (0,)), ((), ())),
            preferred_element_type=jnp.float32)
        base_ref[...] = base
        carry_ref[...] = jnp.zeros_like(carry_ref)
        jt = (jax.lax.broadcasted_iota(jnp.int32, (_NT, _E), 0)
              .astype(jnp.float32) * float(_T))
        teb = jnp.sum((base <= jt).astype(jnp.float32), axis=1,
                      keepdims=True) - 1.0
        te_ref[...] = teb.astype(jnp.int32)

    @pl.when(i > 0)
    def _():
        we = we_ref[...]
        sel = (we > 0.0).astype(jnp.bfloat16)
        ru = jax.lax.broadcasted_iota(jnp.int32, (_ST, _ST), 0)
        rv = jax.lax.broadcasted_iota(jnp.int32, (_ST, _ST), 1)
        tril = (rv < ru).astype(jnp.bfloat16)
        rank = jax.lax.dot_general(
            tril, sel, (((1,), (0,)), ((), ())),
            preferred_element_type=jnp.float32) + carry_ref[...]
        carry_ref[...] += jnp.sum(sel.astype(jnp.float32), axis=0,
                                  keepdims=True)
        pos = (base_ref[...] + rank).astype(jnp.int32)
        selb = we > 0.0
        it = jax.lax.broadcasted_iota(jnp.int32, we.shape, 1)
        ilo = jnp.min(jnp.where(selb, it, _E), axis=-1, keepdims=True)
        ihi = jnp.max(jnp.where(selb, it, -1), axis=-1, keepdims=True)
        dlo_ref[...] = jnp.sum(jnp.where(it == ilo, pos, 0), axis=-1,
                               keepdims=True)
        dhi_ref[...] = jnp.sum(jnp.where(it == ihi, pos, 0), axis=-1,
                               keepdims=True)
        wlo_ref[...] = jnp.sum(jnp.where(it == ilo, we, 0.0), axis=-1,
                               keepdims=True)
        whi_ref[...] = jnp.sum(jnp.where(it == ihi, we, 0.0), axis=-1,
                               keepdims=True)


def _grouped_kernel(te_ref, xg_ref, wg_ref, wu_ref, wd_ref, y_ref):
    xb = xg_ref[...]
    g = jax.lax.dot_general(
        xb, wg_ref[0], (((1,), (1,)), ((), ())),
        preferred_element_type=jnp.float32)
    u = jax.lax.dot_general(
        xb, wu_ref[0], (((1,), (1,)), ((), ())),
        preferred_element_type=jnp.float32)
    a = (g * jax.nn.sigmoid(g)) * u
    y_ref[...] = jax.lax.dot_general(
        a.astype(jnp.bfloat16), wd_ref[0], (((1,), (1,)), ((), ())),
        preferred_element_type=jnp.float32).astype(jnp.bfloat16)


def _combine_kernel(h_ref, x2b_ref, ylo_ref, yhi_ref, wlo_ref, whi_ref,
                    wsg_ref, wsu_ref, wsd_ref, out_ref):
    x2b = x2b_ref[...]
    sg = jax.lax.dot_general(
        x2b, wsg_ref[...], (((1,), (1,)), ((), ())),
        preferred_element_type=jnp.float32)
    su = jax.lax.dot_general(
        x2b, wsu_ref[...], (((1,), (1,)), ((), ())),
        preferred_element_type=jnp.float32)
    ash = (sg * jax.nn.sigmoid(sg)) * su
    osh = jax.lax.dot_general(
        ash.astype(jnp.bfloat16), wsd_ref[...], (((1,), (1,)), ((), ())),
        preferred_element_type=jnp.float32)
    out_ref[...] = (h_ref[...] + osh
                    + wlo_ref[...] * ylo_ref[...].astype(jnp.float32)
                    + whi_ref[...] * yhi_ref[...].astype(jnp.float32))


def _sc_mesh():
    return plsc.VectorSubcoreMesh(core_axis_name="c", subcore_axis_name="s")


def _sc_dispatch(x2v, dlo, dhi):
    @functools.partial(
        pl.kernel,
        out_type=jax.ShapeDtypeStruct((_NTT, _D // 2), jnp.float32),
        mesh=_sc_mesh(),
        scratch_types=[pltpu.VMEM((_CH,), jnp.int32),
                       pltpu.VMEM((_CH, _D // 2), jnp.float32)],
    )
    def k(x2_hbm, dlo_hbm, dhi_hbm, xg_hbm, idx_v, rows_v):
        wid = jax.lax.axis_index("s") * 2 + jax.lax.axis_index("c")
        base = wid * _CH
        pltpu.sync_copy(x2_hbm.at[pl.ds(base, _CH)], rows_v)
        pltpu.sync_copy(dlo_hbm.at[pl.ds(base, _CH)], idx_v)
        pltpu.sync_copy(rows_v, xg_hbm.at[idx_v])
        pltpu.sync_copy(dhi_hbm.at[pl.ds(base, _CH)], idx_v)
        pltpu.sync_copy(rows_v, xg_hbm.at[idx_v])

    return k(x2v, dlo, dhi)


def _sc_combine(yv, dlo, dhi):
    @functools.partial(
        pl.kernel,
        out_type=[jax.ShapeDtypeStruct((_S, _D // 2), jnp.float32),
                  jax.ShapeDtypeStruct((_S, _D // 2), jnp.float32)],
        mesh=_sc_mesh(),
        scratch_types=[pltpu.VMEM((_CH,), jnp.int32),
                       pltpu.VMEM((_CH, _D // 2), jnp.float32),
                       pltpu.SemaphoreType.DMA],
    )
    def k(y_hbm, dlo_hbm, dhi_hbm, ylo_hbm, yhi_hbm, idx_v, rows_v, sem):
        wid = jax.lax.axis_index("s") * 2 + jax.lax.axis_index("c")
        base = wid * _CH
        pltpu.sync_copy(dlo_hbm.at[pl.ds(base, _CH)], idx_v)
        pltpu.async_copy(y_hbm.at[idx_v], rows_v, sem).wait()
        pltpu.sync_copy(rows_v, ylo_hbm.at[pl.ds(base, _CH)])
        pltpu.sync_copy(dhi_hbm.at[pl.ds(base, _CH)], idx_v)
        pltpu.async_copy(y_hbm.at[idx_v], rows_v, sem).wait()
        pltpu.sync_copy(rows_v, yhi_hbm.at[pl.ds(base, _CH)])

    return k(yv, dlo, dhi)


def kernel(hidden_states, position_ids, ln1_w, Wqkv, q_ln_w, k_ln_w, Wo,
           ln2_w, Wr, Wg, Wu, Wd, Wsg, Wsu, Wsd):
    x = hidden_states.reshape(_S, _D)
    pos = position_ids.reshape(_S, 1).astype(jnp.float32)
    inv_freq = (1.0 / (_THETA ** (jnp.arange(0, _HD, 2, dtype=jnp.float32)
                                  / _HD))).reshape(1, _HD // 2)
    ln1 = ln1_w.reshape(1, _D)
    ln2 = ln2_w.reshape(1, _D)
    qln = q_ln_w.reshape(1, _HD)
    kln = k_ln_w.reshape(1, _HD)
    wqkv_bf = Wqkv.astype(jnp.bfloat16)
    wo_bf = Wo.astype(jnp.bfloat16)

    nt = _S // _ST
    qkv, cos, sin = pl.pallas_call(
        _stage1_kernel,
        grid=(nt,),
        in_specs=[
            pl.BlockSpec((_ST, 1), lambda i: (i, 0)),
            pl.BlockSpec((1, _HD // 2), lambda i: (0, 0)),
            pl.BlockSpec((_ST, _D), lambda i: (i, 0)),
            pl.BlockSpec((1, _D), lambda i: (0, 0)),
            pl.BlockSpec((_QKVD, _D), lambda i: (0, 0)),
        ],
        out_specs=[
            pl.BlockSpec((_ST, _QKVD), lambda i: (i, 0)),
            pl.BlockSpec((_ST, _HD // 2), lambda i: (i, 0)),
            pl.BlockSpec((_ST, _HD // 2), lambda i: (i, 0)),
        ],
        out_shape=[
            jax.ShapeDtypeStruct((_S, _QKVD), jnp.float32),
            jax.ShapeDtypeStruct((_S, _HD // 2), jnp.float32),
            jax.ShapeDtypeStruct((_S, _HD // 2), jnp.float32),
        ],
    )(pos, inv_freq, x, ln1, wqkv_bf)

    ctx = pl.pallas_call(
        _attn_kernel,
        grid=(_S // _QT, _H // 2),
        in_specs=[
            pl.BlockSpec((_S, _HD // 2), lambda t, h2: (0, 0)),
            pl.BlockSpec((_S, _HD // 2), lambda t, h2: (0, 0)),
            pl.BlockSpec((1, _HD), lambda t, h2: (0, 0)),
            pl.BlockSpec((1, _HD), lambda t, h2: (0, 0)),
            pl.BlockSpec((_QT, 2 * _HD), lambda t, h2: (t, h2)),
            pl.BlockSpec((_S, 512), lambda t, h2: (0, 2)),
        ],
        out_specs=pl.BlockSpec((_QT, 2 * _HD), lambda t, h2: (t, h2)),
        out_shape=jax.ShapeDtypeStruct((_S, _H * _HD), jnp.float32),
    )(cos, sin, qln, kln, qkv, qkv)

    h, x2b, we, cnt = pl.pallas_call(
        _stage3_kernel,
        grid=(nt,),
        in_specs=[
            pl.BlockSpec((_ST, _D), lambda i: (i, 0)),
            pl.BlockSpec((_ST, _D), lambda i: (i, 0)),
            pl.BlockSpec((_D, _D), lambda i: (0, 0)),
            pl.BlockSpec((1, _D), lambda i: (0, 0)),
            pl.BlockSpec((_E, _D), lambda i: (0, 0)),
        ],
        out_specs=[
            pl.BlockSpec((_ST, _D), lambda i: (i, 0)),
            pl.BlockSpec((_ST, _D), lambda i: (i, 0)),
            pl.BlockSpec((_ST, _E), lambda i: (i, 0)),
            pl.BlockSpec((1, _E), lambda i: (0, 0)),
        ],
        out_shape=[
            jax.ShapeDtypeStruct((_S, _D), jnp.float32),
            jax.ShapeDtypeStruct((_S, _D), jnp.bfloat16),
            jax.ShapeDtypeStruct((_S, _E), jnp.float32),
            jax.ShapeDtypeStruct((1, _E), jnp.float32),
        ],
    )(x, ctx, wo_bf, ln2, Wr)

    dlo, dhi, wlo, whi, te = pl.pallas_call(
        _meta_kernel,
        grid=(nt + 1,),
        in_specs=[
            pl.BlockSpec((1, _E), lambda i: (0, 0)),
            pl.BlockSpec((_ST, _E),
                         lambda i: (jnp.maximum(i - 1, 0), 0)),
        ],
        out_specs=[
            pl.BlockSpec((_ST, 1), lambda i: (jnp.maximum(i - 1, 0), 0)),
            pl.BlockSpec((_ST, 1), lambda i: (jnp.maximum(i - 1, 0), 0)),
            pl.BlockSpec((_ST, 1), lambda i: (jnp.maximum(i - 1, 0), 0)),
            pl.BlockSpec((_ST, 1), lambda i: (jnp.maximum(i - 1, 0), 0)),
            pl.BlockSpec((_NT, 1), lambda i: (0, 0)),
        ],
        out_shape=[
            jax.ShapeDtypeStruct((_S, 1), jnp.int32),
            jax.ShapeDtypeStruct((_S, 1), jnp.int32),
            jax.ShapeDtypeStruct((_S, 1), jnp.float32),
            jax.ShapeDtypeStruct((_S, 1), jnp.float32),
            jax.ShapeDtypeStruct((_NT, 1), jnp.int32),
        ],
        scratch_shapes=[pltpu.VMEM((1, _E), jnp.float32),
                        pltpu.VMEM((1, _E), jnp.float32)],
    )(cnt, we)

    x2v = jax.lax.bitcast_convert_type(
        x2b.reshape(_S, _D // 2, 2), jnp.float32)
    dlo1 = dlo.reshape(_S)
    dhi1 = dhi.reshape(_S)
    xgv = _sc_dispatch(x2v, dlo1, dhi1)
    xg = jax.lax.bitcast_convert_type(xgv, jnp.bfloat16).reshape(_NTT, _D)

    wg8 = Wg.astype(jnp.bfloat16)
    wu8 = Wu.astype(jnp.bfloat16)
    wd8 = Wd.astype(jnp.bfloat16)

    y = pl.pallas_call(
        _grouped_kernel,
        grid_spec=pltpu.PrefetchScalarGridSpec(
            num_scalar_prefetch=1,
            grid=(_NT,),
            in_specs=[
                pl.BlockSpec((_T, _D), lambda j, te: (j, 0)),
                pl.BlockSpec((1, _DFF, _D), lambda j, te: (te[j], 0, 0)),
                pl.BlockSpec((1, _DFF, _D), lambda j, te: (te[j], 0, 0)),
                pl.BlockSpec((1, _D, _DFF), lambda j, te: (te[j], 0, 0)),
            ],
            out_specs=pl.BlockSpec((_T, _D), lambda j, te: (j, 0)),
        ),
        out_shape=jax.ShapeDtypeStruct((_NTT, _D), jnp.bfloat16),
    )(te.reshape(_NT), xg, wg8, wu8, wd8)

    yv = jax.lax.bitcast_convert_type(
        y.reshape(_NTT, _D // 2, 2), jnp.float32)
    ylov, yhiv = _sc_combine(yv, dlo1, dhi1)
    ylo = jax.lax.bitcast_convert_type(ylov, jnp.bfloat16).reshape(_S, _D)
    yhi = jax.lax.bitcast_convert_type(yhiv, jnp.bfloat16).reshape(_S, _D)

    out = pl.pallas_call(
        _combine_kernel,
        grid=(nt,),
        in_specs=[
            pl.BlockSpec((_ST, _D), lambda i: (i, 0)),
            pl.BlockSpec((_ST, _D), lambda i: (i, 0)),
            pl.BlockSpec((_ST, _D), lambda i: (i, 0)),
            pl.BlockSpec((_ST, _D), lambda i: (i, 0)),
            pl.BlockSpec((_ST, 1), lambda i: (i, 0)),
            pl.BlockSpec((_ST, 1), lambda i: (i, 0)),
            pl.BlockSpec((_DFF, _D), lambda i: (0, 0)),
            pl.BlockSpec((_DFF, _D), lambda i: (0, 0)),
            pl.BlockSpec((_D, _DFF), lambda i: (0, 0)),
        ],
        out_specs=pl.BlockSpec((_ST, _D), lambda i: (i, 0)),
        out_shape=jax.ShapeDtypeStruct((_S, _D), jnp.float32),
    )(h, x2b, ylo, yhi, wlo, whi, Wsg.astype(jnp.bfloat16),
      Wsu.astype(jnp.bfloat16), Wsd.astype(jnp.bfloat16))

    return out.reshape(1, _S, _D)


# speedup vs baseline: 2.8465x; 2.8465x over previous
import jax
import jax.numpy as jnp
from jax.experimental import pallas as pl
from jax.experimental.pallas import tpu as pltpu

_S, _D = 2048, 1024
_H, _HKV, _HD = 16, 4, 64
_E, _DFF = 8, 512
_G = _H // _HKV
_QKVD = (_H + 2 * _HKV) * _HD
_EPS = 1e-6
_THETA = 10000.0
_ST = 256
_QT = 512


def _stage1_kernel(pos_ref, inv_ref, x_ref, w1_ref, wqkv_ref,
                   qkv_ref, cos_ref, sin_ref):
    x = x_ref[...]
    v = jnp.mean(x * x, axis=-1, keepdims=True)
    xn = w1_ref[...] * (x * jax.lax.rsqrt(v + _EPS))
    qkv_ref[...] = jax.lax.dot_general(
        xn.astype(jnp.bfloat16), wqkv_ref[...],
        (((1,), (1,)), ((), ())), preferred_element_type=jnp.float32)
    f = pos_ref[...] * inv_ref[...]
    cos_ref[...] = jnp.cos(f)
    sin_ref[...] = jnp.sin(f)


def _rope(xn, c, s):
    x1 = xn[:, : _HD // 2]
    x2 = xn[:, _HD // 2:]
    return jnp.concatenate([x1 * c - x2 * s, x2 * c + x1 * s], axis=-1)


def _attn_kernel(cos_ref, sin_ref, qln_ref, kln_ref, q_ref, kv_ref,
                 ctx_ref, kr_ref, vb_ref):
    h2 = pl.program_id(0)
    t = pl.program_id(1)
    g2 = h2 // 2
    cq = cos_ref[pl.ds(t * _QT, _QT), :]
    sq = sin_ref[pl.ds(t * _QT, _QT), :]

    @pl.when(t == 0)
    def _():
        kv = kv_ref[...]
        ksel = jnp.where(
            g2 < 2,
            jnp.where(g2 == 0, kv[:, 0:_HD], kv[:, _HD:2 * _HD]),
            jnp.where(g2 == 2, kv[:, 2 * _HD:3 * _HD],
                      kv[:, 3 * _HD:4 * _HD]))
        vsel = jnp.where(
            g2 < 2,
            jnp.where(g2 == 0, kv[:, 4 * _HD:5 * _HD],
                      kv[:, 5 * _HD:6 * _HD]),
            jnp.where(g2 == 2, kv[:, 6 * _HD:7 * _HD],
                      kv[:, 7 * _HD:8 * _HD]))
        kvar = jnp.mean(ksel * ksel, axis=-1, keepdims=True)
        kn = kln_ref[...] * (ksel * jax.lax.rsqrt(kvar + _EPS))
        kr_ref[...] = _rope(kn, cos_ref[...], sin_ref[...]).astype(
            jnp.bfloat16)
        vb_ref[...] = vsel.astype(jnp.bfloat16)

    kr = kr_ref[...]
    vb = vb_ref[...]
    qpair = q_ref[...]
    qs = qln_ref[...] * 0.125

    def head(q):
        qv = jnp.mean(q * q, axis=-1, keepdims=True)
        qn = qs * (q * jax.lax.rsqrt(qv + _EPS))
        qr = _rope(qn, cq, sq).astype(jnp.bfloat16)
        scores = jax.lax.dot_general(
            qr, kr, (((1,), (1,)), ((), ())),
            preferred_element_type=jnp.float32)
        p = jnp.exp(scores)
        s = jnp.sum(p, axis=-1, keepdims=True)
        ctx = jax.lax.dot_general(
            p.astype(jnp.bfloat16), vb, (((1,), (0,)), ((), ())),
            preferred_element_type=jnp.float32)
        return ctx / s

    ctx_ref[...] = jnp.concatenate(
        [head(qpair[:, :_HD]), head(qpair[:, _HD:])], axis=-1)


def _stage3_kernel(res_ref, ctx_ref, wo_ref, ln2_ref, wr_ref,
                   h_ref, x2b_ref, we_ref):
    ao = jax.lax.dot_general(
        ctx_ref[...].astype(jnp.bfloat16), wo_ref[...],
        (((1,), (1,)), ((), ())), preferred_element_type=jnp.float32)
    h = res_ref[...] + ao
    h_ref[...] = h
    v = jnp.mean(h * h, axis=-1, keepdims=True)
    x2 = ln2_ref[...] * (h * jax.lax.rsqrt(v + _EPS))
    x2b_ref[...] = x2.astype(jnp.bfloat16)
    logits = jax.lax.dot_general(
        x2, wr_ref[...], (((1,), (1,)), ((), ())),
        preferred_element_type=jnp.float32)
    lm = jnp.max(logits, axis=-1, keepdims=True)
    el = jnp.exp(logits - lm)
    p = el / jnp.sum(el, axis=-1, keepdims=True)
    it = jax.lax.broadcasted_iota(jnp.int32, p.shape, 1)
    m1 = jnp.max(p, axis=-1, keepdims=True)
    i1 = jnp.min(jnp.where(p == m1, it, _E), axis=-1, keepdims=True)
    p2 = jnp.where(it == i1, -1.0, p)
    m2 = jnp.max(p2, axis=-1, keepdims=True)
    i2 = jnp.min(jnp.where(p2 == m2, it, _E), axis=-1, keepdims=True)
    we = jnp.where(it == i1, m1, 0.0) + jnp.where(it == i2, m2, 0.0)
    we_ref[...] = we / (m1 + m2)


def _moe_kernel(x2b_ref, h_ref, we_ref, wg_ref, wu_ref, wd_ref, out_ref):
    c = pl.program_id(0)
    x2b = x2b_ref[...]
    g = jax.lax.dot_general(
        x2b, wg_ref[0], (((1,), (1,)), ((), ())),
        preferred_element_type=jnp.float32)
    u = jax.lax.dot_general(
        x2b, wu_ref[0], (((1,), (1,)), ((), ())),
        preferred_element_type=jnp.float32)
    a = (g * jax.nn.sigmoid(g)) * u
    it = jax.lax.broadcasted_iota(jnp.int32, we_ref.shape, 1)
    w = jnp.sum(jnp.where(it == jnp.minimum(c, _E - 1), we_ref[...], 0.0),
                axis=-1, keepdims=True)
    a = a * jnp.where(c == _E, 1.0, w)
    part = jax.lax.dot_general(
        a.astype(jnp.bfloat16), wd_ref[0], (((1,), (1,)), ((), ())),
        preferred_element_type=jnp.float32)

    @pl.when(c == 0)
    def _():
        out_ref[...] = h_ref[...]

    out_ref[...] += part


def kernel(hidden_states, position_ids, ln1_w, Wqkv, q_ln_w, k_ln_w, Wo,
           ln2_w, Wr, Wg, Wu, Wd, Wsg, Wsu, Wsd):
    x = hidden_states.reshape(_S, _D)
    pos = position_ids.reshape(_S, 1).astype(jnp.float32)
    inv_freq = (1.0 / (_THETA ** (jnp.arange(0, _HD, 2, dtype=jnp.float32)
                                  / _HD))).reshape(1, _HD // 2)
    ln1 = ln1_w.reshape(1, _D)
    ln2 = ln2_w.reshape(1, _D)
    qln = q_ln_w.reshape(1, _HD)
    kln = k_ln_w.reshape(1, _HD)
    wqkv_bf = Wqkv.astype(jnp.bfloat16)
    wo_bf = Wo.astype(jnp.bfloat16)

    nt = _S // _ST
    qkv, cos, sin = pl.pallas_call(
        _stage1_kernel,
        grid=(nt,),
        in_specs=[
            pl.BlockSpec((_ST, 1), lambda i: (i, 0)),
            pl.BlockSpec((1, _HD // 2), lambda i: (0, 0)),
            pl.BlockSpec((_ST, _D), lambda i: (i, 0)),
            pl.BlockSpec((1, _D), lambda i: (0, 0)),
            pl.BlockSpec((_QKVD, _D), lambda i: (0, 0)),
        ],
        out_specs=[
            pl.BlockSpec((_ST, _QKVD), lambda i: (i, 0)),
            pl.BlockSpec((_ST, _HD // 2), lambda i: (i, 0)),
            pl.BlockSpec((_ST, _HD // 2), lambda i: (i, 0)),
        ],
        out_shape=[
            jax.ShapeDtypeStruct((_S, _QKVD), jnp.float32),
            jax.ShapeDtypeStruct((_S, _HD // 2), jnp.float32),
            jax.ShapeDtypeStruct((_S, _HD // 2), jnp.float32),
        ],
    )(pos, inv_freq, x, ln1, wqkv_bf)

    ctx = pl.pallas_call(
        _attn_kernel,
        grid=(_H // 2, _S // _QT),
        in_specs=[
            pl.BlockSpec((_S, _HD // 2), lambda h2, t: (0, 0)),
            pl.BlockSpec((_S, _HD // 2), lambda h2, t: (0, 0)),
            pl.BlockSpec((1, _HD), lambda h2, t: (0, 0)),
            pl.BlockSpec((1, _HD), lambda h2, t: (0, 0)),
            pl.BlockSpec((_QT, 2 * _HD), lambda h2, t: (t, h2)),
            pl.BlockSpec((_S, 512), lambda h2, t: (0, 2)),
        ],
        out_specs=pl.BlockSpec((_QT, 2 * _HD), lambda h2, t: (t, h2)),
        out_shape=jax.ShapeDtypeStruct((_S, _H * _HD), jnp.float32),
        scratch_shapes=[pltpu.VMEM((_S, _HD), jnp.bfloat16),
                        pltpu.VMEM((_S, _HD), jnp.bfloat16)],
    )(cos, sin, qln, kln, qkv, qkv)

    h, x2b, we = pl.pallas_call(
        _stage3_kernel,
        grid=(nt,),
        in_specs=[
            pl.BlockSpec((_ST, _D), lambda i: (i, 0)),
            pl.BlockSpec((_ST, _D), lambda i: (i, 0)),
            pl.BlockSpec((_D, _D), lambda i: (0, 0)),
            pl.BlockSpec((1, _D), lambda i: (0, 0)),
            pl.BlockSpec((_E, _D), lambda i: (0, 0)),
        ],
        out_specs=[
            pl.BlockSpec((_ST, _D), lambda i: (i, 0)),
            pl.BlockSpec((_ST, _D), lambda i: (i, 0)),
            pl.BlockSpec((_ST, _E), lambda i: (i, 0)),
        ],
        out_shape=[
            jax.ShapeDtypeStruct((_S, _D), jnp.float32),
            jax.ShapeDtypeStruct((_S, _D), jnp.bfloat16),
            jax.ShapeDtypeStruct((_S, _E), jnp.float32),
        ],
    )(x, ctx, wo_bf, ln2, Wr)

    wg_cat = jnp.concatenate(
        [Wg, Wsg.reshape(1, _DFF, _D)], axis=0).astype(jnp.bfloat16)
    wu_cat = jnp.concatenate(
        [Wu, Wsu.reshape(1, _DFF, _D)], axis=0).astype(jnp.bfloat16)
    wd_cat = jnp.concatenate(
        [Wd, Wsd.reshape(1, _D, _DFF)], axis=0).astype(jnp.bfloat16)

    out = pl.pallas_call(
        _moe_kernel,
        grid=(_E + 1,),
        in_specs=[
            pl.BlockSpec((_S, _D), lambda c: (0, 0)),
            pl.BlockSpec((_S, _D), lambda c: (0, 0)),
            pl.BlockSpec((_S, _E), lambda c: (0, 0)),
            pl.BlockSpec((1, _DFF, _D), lambda c: (c, 0, 0)),
            pl.BlockSpec((1, _DFF, _D), lambda c: (c, 0, 0)),
            pl.BlockSpec((1, _D, _DFF), lambda c: (c, 0, 0)),
        ],
        out_specs=pl.BlockSpec((_S, _D), lambda c: (0, 0)),
        out_shape=jax.ShapeDtypeStruct((_S, _D), jnp.float32),
    )(x2b, h, we, wg_cat, wu_cat, wd_cat)

    return out.reshape(1, _S, _D)


# speedup vs baseline: 2.9061x; 1.0209x over previous
import jax
import jax.numpy as jnp
from jax.experimental import pallas as pl
from jax.experimental.pallas import tpu as pltpu

_S, _D = 2048, 1024
_H, _HKV, _HD = 16, 4, 64
_E, _DFF = 8, 512
_G = _H // _HKV
_QKVD = (_H + 2 * _HKV) * _HD
_EPS = 1e-6
_THETA = 10000.0
_ST = 256
_QT = 1024


def _stage1_kernel(pos_ref, inv_ref, x_ref, w1_ref, wqkv_ref,
                   qkv_ref, cos_ref, sin_ref):
    x = x_ref[...]
    v = jnp.mean(x * x, axis=-1, keepdims=True)
    xn = w1_ref[...] * (x * jax.lax.rsqrt(v + _EPS))
    qkv_ref[...] = jax.lax.dot_general(
        xn.astype(jnp.bfloat16), wqkv_ref[...],
        (((1,), (1,)), ((), ())), preferred_element_type=jnp.float32)
    f = pos_ref[...] * inv_ref[...]
    cos_ref[...] = jnp.cos(f)
    sin_ref[...] = jnp.sin(f)


def _rope(xn, c, s):
    x1 = xn[:, : _HD // 2]
    x2 = xn[:, _HD // 2:]
    return jnp.concatenate([x1 * c - x2 * s, x2 * c + x1 * s], axis=-1)


def _attn_kernel(cos_ref, sin_ref, qln_ref, kln_ref, q_ref, kv_ref,
                 ctx_ref, kr_ref, vb_ref):
    h2 = pl.program_id(0)
    t = pl.program_id(1)
    g2 = h2 // 2
    cq = cos_ref[pl.ds(t * _QT, _QT), :]
    sq = sin_ref[pl.ds(t * _QT, _QT), :]

    @pl.when(t == 0)
    def _():
        kv = kv_ref[...]
        ksel = jnp.where(
            g2 < 2,
            jnp.where(g2 == 0, kv[:, 0:_HD], kv[:, _HD:2 * _HD]),
            jnp.where(g2 == 2, kv[:, 2 * _HD:3 * _HD],
                      kv[:, 3 * _HD:4 * _HD]))
        vsel = jnp.where(
            g2 < 2,
            jnp.where(g2 == 0, kv[:, 4 * _HD:5 * _HD],
                      kv[:, 5 * _HD:6 * _HD]),
            jnp.where(g2 == 2, kv[:, 6 * _HD:7 * _HD],
                      kv[:, 7 * _HD:8 * _HD]))
        kvar = jnp.mean(ksel * ksel, axis=-1, keepdims=True)
        kn = kln_ref[...] * (ksel * jax.lax.rsqrt(kvar + _EPS))
        kr_ref[...] = _rope(kn, cos_ref[...], sin_ref[...]).astype(
            jnp.bfloat16)
        vb_ref[...] = vsel.astype(jnp.bfloat16)

    kr = kr_ref[...]
    vb = vb_ref[...]
    qpair = q_ref[...]
    qs = qln_ref[...] * 0.125

    def head(q):
        qv = jnp.mean(q * q, axis=-1, keepdims=True)
        qn = qs * (q * jax.lax.rsqrt(qv + _EPS))
        qr = _rope(qn, cq, sq).astype(jnp.bfloat16)
        scores = jax.lax.dot_general(
            qr, kr, (((1,), (1,)), ((), ())),
            preferred_element_type=jnp.float32)
        p = jnp.exp(scores)
        s = jnp.sum(p, axis=-1, keepdims=True)
        ctx = jax.lax.dot_general(
            p.astype(jnp.bfloat16), vb, (((1,), (0,)), ((), ())),
            preferred_element_type=jnp.float32)
        return ctx / s

    ctx_ref[...] = jnp.concatenate(
        [head(qpair[:, :_HD]), head(qpair[:, _HD:])], axis=-1)


def _stage3_kernel(res_ref, ctx_ref, wo_ref, ln2_ref, wr_ref,
                   h_ref, x2b_ref, we_ref):
    ao = jax.lax.dot_general(
        ctx_ref[...].astype(jnp.bfloat16), wo_ref[...],
        (((1,), (1,)), ((), ())), preferred_element_type=jnp.float32)
    h = res_ref[...] + ao
    h_ref[...] = h
    v = jnp.mean(h * h, axis=-1, keepdims=True)
    x2 = ln2_ref[...] * (h * jax.lax.rsqrt(v + _EPS))
    x2b_ref[...] = x2.astype(jnp.bfloat16)
    logits = jax.lax.dot_general(
        x2, wr_ref[...], (((1,), (1,)), ((), ())),
        preferred_element_type=jnp.float32)
    lm = jnp.max(logits, axis=-1, keepdims=True)
    el = jnp.exp(logits - lm)
    p = el / jnp.sum(el, axis=-1, keepdims=True)
    it = jax.lax.broadcasted_iota(jnp.int32, p.shape, 1)
    m1 = jnp.max(p, axis=-1, keepdims=True)
    i1 = jnp.min(jnp.where(p == m1, it, _E), axis=-1, keepdims=True)
    p2 = jnp.where(it == i1, -1.0, p)
    m2 = jnp.max(p2, axis=-1, keepdims=True)
    i2 = jnp.min(jnp.where(p2 == m2, it, _E), axis=-1, keepdims=True)
    we = jnp.where(it == i1, m1, 0.0) + jnp.where(it == i2, m2, 0.0)
    we_ref[...] = we / (m1 + m2)


def _moe_kernel(x2b_ref, h_ref, we_ref, wg_ref, wu_ref, wd_ref, out_ref):
    c = pl.program_id(0)
    x2b = x2b_ref[...]
    g = jax.lax.dot_general(
        x2b, wg_ref[0], (((1,), (1,)), ((), ())),
        preferred_element_type=jnp.float32)
    u = jax.lax.dot_general(
        x2b, wu_ref[0], (((1,), (1,)), ((), ())),
        preferred_element_type=jnp.float32)
    a = (g * jax.nn.sigmoid(g)) * u
    it = jax.lax.broadcasted_iota(jnp.int32, we_ref.shape, 1)
    w = jnp.sum(jnp.where(it == jnp.minimum(c, _E - 1), we_ref[...], 0.0),
                axis=-1, keepdims=True)
    a = a * jnp.where(c == _E, 1.0, w)
    part = jax.lax.dot_general(
        a.astype(jnp.bfloat16), wd_ref[0], (((1,), (1,)), ((), ())),
        preferred_element_type=jnp.float32)

    @pl.when(c == 0)
    def _():
        out_ref[...] = h_ref[...]

    out_ref[...] += part


def kernel(hidden_states, position_ids, ln1_w, Wqkv, q_ln_w, k_ln_w, Wo,
           ln2_w, Wr, Wg, Wu, Wd, Wsg, Wsu, Wsd):
    x = hidden_states.reshape(_S, _D)
    pos = position_ids.reshape(_S, 1).astype(jnp.float32)
    inv_freq = (1.0 / (_THETA ** (jnp.arange(0, _HD, 2, dtype=jnp.float32)
                                  / _HD))).reshape(1, _HD // 2)
    ln1 = ln1_w.reshape(1, _D)
    ln2 = ln2_w.reshape(1, _D)
    qln = q_ln_w.reshape(1, _HD)
    kln = k_ln_w.reshape(1, _HD)
    wqkv_bf = Wqkv.astype(jnp.bfloat16)
    wo_bf = Wo.astype(jnp.bfloat16)

    nt = _S // _ST
    qkv, cos, sin = pl.pallas_call(
        _stage1_kernel,
        grid=(nt,),
        in_specs=[
            pl.BlockSpec((_ST, 1), lambda i: (i, 0)),
            pl.BlockSpec((1, _HD // 2), lambda i: (0, 0)),
            pl.BlockSpec((_ST, _D), lambda i: (i, 0)),
            pl.BlockSpec((1, _D), lambda i: (0, 0)),
            pl.BlockSpec((_QKVD, _D), lambda i: (0, 0)),
        ],
        out_specs=[
            pl.BlockSpec((_ST, _QKVD), lambda i: (i, 0)),
            pl.BlockSpec((_ST, _HD // 2), lambda i: (i, 0)),
            pl.BlockSpec((_ST, _HD // 2), lambda i: (i, 0)),
        ],
        out_shape=[
            jax.ShapeDtypeStruct((_S, _QKVD), jnp.float32),
            jax.ShapeDtypeStruct((_S, _HD // 2), jnp.float32),
            jax.ShapeDtypeStruct((_S, _HD // 2), jnp.float32),
        ],
    )(pos, inv_freq, x, ln1, wqkv_bf)

    ctx = pl.pallas_call(
        _attn_kernel,
        grid=(_H // 2, _S // _QT),
        in_specs=[
            pl.BlockSpec((_S, _HD // 2), lambda h2, t: (0, 0)),
            pl.BlockSpec((_S, _HD // 2), lambda h2, t: (0, 0)),
            pl.BlockSpec((1, _HD), lambda h2, t: (0, 0)),
            pl.BlockSpec((1, _HD), lambda h2, t: (0, 0)),
            pl.BlockSpec((_QT, 2 * _HD), lambda h2, t: (t, h2)),
            pl.BlockSpec((_S, 512), lambda h2, t: (0, 2)),
        ],
        out_specs=pl.BlockSpec((_QT, 2 * _HD), lambda h2, t: (t, h2)),
        out_shape=jax.ShapeDtypeStruct((_S, _H * _HD), jnp.float32),
        scratch_shapes=[pltpu.VMEM((_S, _HD), jnp.bfloat16),
                        pltpu.VMEM((_S, _HD), jnp.bfloat16)],
    )(cos, sin, qln, kln, qkv, qkv)

    h, x2b, we = pl.pallas_call(
        _stage3_kernel,
        grid=(nt,),
        in_specs=[
            pl.BlockSpec((_ST, _D), lambda i: (i, 0)),
            pl.BlockSpec((_ST, _D), lambda i: (i, 0)),
            pl.BlockSpec((_D, _D), lambda i: (0, 0)),
            pl.BlockSpec((1, _D), lambda i: (0, 0)),
            pl.BlockSpec((_E, _D), lambda i: (0, 0)),
        ],
        out_specs=[
            pl.BlockSpec((_ST, _D), lambda i: (i, 0)),
            pl.BlockSpec((_ST, _D), lambda i: (i, 0)),
            pl.BlockSpec((_ST, _E), lambda i: (i, 0)),
        ],
        out_shape=[
            jax.ShapeDtypeStruct((_S, _D), jnp.float32),
            jax.ShapeDtypeStruct((_S, _D), jnp.bfloat16),
            jax.ShapeDtypeStruct((_S, _E), jnp.float32),
        ],
    )(x, ctx, wo_bf, ln2, Wr)

    wg_cat = jnp.concatenate(
        [Wg, Wsg.reshape(1, _DFF, _D)], axis=0).astype(jnp.bfloat16)
    wu_cat = jnp.concatenate(
        [Wu, Wsu.reshape(1, _DFF, _D)], axis=0).astype(jnp.bfloat16)
    wd_cat = jnp.concatenate(
        [Wd, Wsd.reshape(1, _D, _DFF)], axis=0).astype(jnp.bfloat16)

    out = pl.pallas_call(
        _moe_kernel,
        grid=(_E + 1,),
        in_specs=[
            pl.BlockSpec((_S, _D), lambda c: (0, 0)),
            pl.BlockSpec((_S, _D), lambda c: (0, 0)),
            pl.BlockSpec((_S, _E), lambda c: (0, 0)),
            pl.BlockSpec((1, _DFF, _D), lambda c: (c, 0, 0)),
            pl.BlockSpec((1, _DFF, _D), lambda c: (c, 0, 0)),
            pl.BlockSpec((1, _D, _DFF), lambda c: (c, 0, 0)),
        ],
        out_specs=pl.BlockSpec((_S, _D), lambda c: (0, 0)),
        out_shape=jax.ShapeDtypeStruct((_S, _D), jnp.float32),
    )(x2b, h, we, wg_cat, wu_cat, wd_cat)

    return out.reshape(1, _S, _D)


# speedup vs baseline: 3.2816x; 1.1292x over previous
import jax
import jax.numpy as jnp
from jax.experimental import pallas as pl
from jax.experimental.pallas import tpu as pltpu

_S, _D = 2048, 1024
_H, _HKV, _HD = 16, 4, 64
_E, _DFF = 8, 512
_G = _H // _HKV
_QKVD = (_H + 2 * _HKV) * _HD
_EPS = 1e-6
_THETA = 10000.0
_ST = 256
_QT = 1024


def _stage1_kernel(pos_ref, inv_ref, x_ref, w1_ref, wqkv_ref,
                   qkv_ref, cos_ref, sin_ref):
    x = x_ref[...]
    v = jnp.mean(x * x, axis=-1, keepdims=True)
    xn = w1_ref[...] * (x * jax.lax.rsqrt(v + _EPS))
    qkv_ref[...] = jax.lax.dot_general(
        xn.astype(jnp.bfloat16), wqkv_ref[...],
        (((1,), (1,)), ((), ())), preferred_element_type=jnp.float32)
    f = pos_ref[...] * inv_ref[...]
    cos_ref[...] = jnp.cos(f)
    sin_ref[...] = jnp.sin(f)


def _rope(xn, c, s):
    x1 = xn[:, : _HD // 2]
    x2 = xn[:, _HD // 2:]
    return jnp.concatenate([x1 * c - x2 * s, x2 * c + x1 * s], axis=-1)


def _attn_kernel(cos_ref, sin_ref, qln_ref, kln_ref, q_ref, kv_ref,
                 ctx_ref, kr_ref, vb_ref):
    h2 = pl.program_id(0)
    t = pl.program_id(1)
    g2 = h2 // 2
    cq = cos_ref[pl.ds(t * _QT, _QT), :]
    sq = sin_ref[pl.ds(t * _QT, _QT), :]

    @pl.when(t == 0)
    def _():
        kv = kv_ref[...]
        ksel = jnp.where(
            g2 < 2,
            jnp.where(g2 == 0, kv[:, 0:_HD], kv[:, _HD:2 * _HD]),
            jnp.where(g2 == 2, kv[:, 2 * _HD:3 * _HD],
                      kv[:, 3 * _HD:4 * _HD]))
        vsel = jnp.where(
            g2 < 2,
            jnp.where(g2 == 0, kv[:, 4 * _HD:5 * _HD],
                      kv[:, 5 * _HD:6 * _HD]),
            jnp.where(g2 == 2, kv[:, 6 * _HD:7 * _HD],
                      kv[:, 7 * _HD:8 * _HD]))
        kvar = jnp.mean(ksel * ksel, axis=-1, keepdims=True)
        kn = kln_ref[...] * (ksel * jax.lax.rsqrt(kvar + _EPS))
        kr_ref[...] = _rope(kn, cos_ref[...], sin_ref[...]).astype(
            jnp.bfloat16)
        vb_ref[...] = vsel.astype(jnp.bfloat16)

    kr = kr_ref[...]
    vb = vb_ref[...]
    qpair = q_ref[...]
    qs = qln_ref[...] * 0.125

    def head(q):
        qv = jnp.mean(q * q, axis=-1, keepdims=True)
        qn = qs * (q * jax.lax.rsqrt(qv + _EPS))
        qr = _rope(qn, cq, sq).astype(jnp.bfloat16)
        scores = jax.lax.dot_general(
            qr, kr, (((1,), (1,)), ((), ())),
            preferred_element_type=jnp.float32)
        p = jnp.exp(scores)
        s = jnp.sum(p, axis=-1, keepdims=True)
        ctx = jax.lax.dot_general(
            p.astype(jnp.bfloat16), vb, (((1,), (0,)), ((), ())),
            preferred_element_type=jnp.float32)
        return ctx / s

    ctx_ref[...] = jnp.concatenate(
        [head(qpair[:, :_HD]), head(qpair[:, _HD:])], axis=-1)


def _stage3_kernel(res_ref, ctx_ref, wo_ref, ln2_ref, wr_ref,
                   h_ref, x2b_ref, we_ref):
    ao = jax.lax.dot_general(
        ctx_ref[...].astype(jnp.bfloat16), wo_ref[...],
        (((1,), (1,)), ((), ())), preferred_element_type=jnp.float32)
    h = res_ref[...] + ao
    h_ref[...] = h
    v = jnp.mean(h * h, axis=-1, keepdims=True)
    x2 = ln2_ref[...] * (h * jax.lax.rsqrt(v + _EPS))
    x2b_ref[...] = x2.astype(jnp.bfloat16)
    logits = jax.lax.dot_general(
        x2, wr_ref[...], (((1,), (1,)), ((), ())),
        preferred_element_type=jnp.float32)
    lm = jnp.max(logits, axis=-1, keepdims=True)
    el = jnp.exp(logits - lm)
    p = el / jnp.sum(el, axis=-1, keepdims=True)
    it = jax.lax.broadcasted_iota(jnp.int32, p.shape, 1)
    m1 = jnp.max(p, axis=-1, keepdims=True)
    i1 = jnp.min(jnp.where(p == m1, it, _E), axis=-1, keepdims=True)
    p2 = jnp.where(it == i1, -1.0, p)
    m2 = jnp.max(p2, axis=-1, keepdims=True)
    i2 = jnp.min(jnp.where(p2 == m2, it, _E), axis=-1, keepdims=True)
    we = jnp.where(it == i1, m1, 0.0) + jnp.where(it == i2, m2, 0.0)
    we_ref[...] = we / (m1 + m2)


def _moe_kernel(x2b_ref, h_ref, we_ref, wg_ref, wu_ref, wd_ref,
                wsg_ref, wsu_ref, wsd_ref, out_ref):
    c = pl.program_id(0)
    shared = c == _E
    x2b = x2b_ref[...]
    wg = jnp.where(shared, wsg_ref[...], wg_ref[0]).astype(jnp.bfloat16)
    wu = jnp.where(shared, wsu_ref[...], wu_ref[0]).astype(jnp.bfloat16)
    wd = jnp.where(shared, wsd_ref[...], wd_ref[0]).astype(jnp.bfloat16)
    g = jax.lax.dot_general(
        x2b, wg, (((1,), (1,)), ((), ())),
        preferred_element_type=jnp.float32)
    u = jax.lax.dot_general(
        x2b, wu, (((1,), (1,)), ((), ())),
        preferred_element_type=jnp.float32)
    a = (g * jax.nn.sigmoid(g)) * u
    it = jax.lax.broadcasted_iota(jnp.int32, we_ref.shape, 1)
    w = jnp.sum(jnp.where(it == jnp.minimum(c, _E - 1), we_ref[...], 0.0),
                axis=-1, keepdims=True)
    a = a * jnp.where(shared, 1.0, w)
    part = jax.lax.dot_general(
        a.astype(jnp.bfloat16), wd, (((1,), (1,)), ((), ())),
        preferred_element_type=jnp.float32)

    @pl.when(c == 0)
    def _():
        out_ref[...] = h_ref[...]

    out_ref[...] += part


def kernel(hidden_states, position_ids, ln1_w, Wqkv, q_ln_w, k_ln_w, Wo,
           ln2_w, Wr, Wg, Wu, Wd, Wsg, Wsu, Wsd):
    x = hidden_states.reshape(_S, _D)
    pos = position_ids.reshape(_S, 1).astype(jnp.float32)
    inv_freq = (1.0 / (_THETA ** (jnp.arange(0, _HD, 2, dtype=jnp.float32)
                                  / _HD))).reshape(1, _HD // 2)
    ln1 = ln1_w.reshape(1, _D)
    ln2 = ln2_w.reshape(1, _D)
    qln = q_ln_w.reshape(1, _HD)
    kln = k_ln_w.reshape(1, _HD)
    wqkv_bf = Wqkv.astype(jnp.bfloat16)
    wo_bf = Wo.astype(jnp.bfloat16)

    nt = _S // _ST
    qkv, cos, sin = pl.pallas_call(
        _stage1_kernel,
        grid=(nt,),
        in_specs=[
            pl.BlockSpec((_ST, 1), lambda i: (i, 0)),
            pl.BlockSpec((1, _HD // 2), lambda i: (0, 0)),
            pl.BlockSpec((_ST, _D), lambda i: (i, 0)),
            pl.BlockSpec((1, _D), lambda i: (0, 0)),
            pl.BlockSpec((_QKVD, _D), lambda i: (0, 0)),
        ],
        out_specs=[
            pl.BlockSpec((_ST, _QKVD), lambda i: (i, 0)),
            pl.BlockSpec((_ST, _HD // 2), lambda i: (i, 0)),
            pl.BlockSpec((_ST, _HD // 2), lambda i: (i, 0)),
        ],
        out_shape=[
            jax.ShapeDtypeStruct((_S, _QKVD), jnp.float32),
            jax.ShapeDtypeStruct((_S, _HD // 2), jnp.float32),
            jax.ShapeDtypeStruct((_S, _HD // 2), jnp.float32),
        ],
    )(pos, inv_freq, x, ln1, wqkv_bf)

    ctx = pl.pallas_call(
        _attn_kernel,
        grid=(_H // 2, _S // _QT),
        in_specs=[
            pl.BlockSpec((_S, _HD // 2), lambda h2, t: (0, 0)),
            pl.BlockSpec((_S, _HD // 2), lambda h2, t: (0, 0)),
            pl.BlockSpec((1, _HD), lambda h2, t: (0, 0)),
            pl.BlockSpec((1, _HD), lambda h2, t: (0, 0)),
            pl.BlockSpec((_QT, 2 * _HD), lambda h2, t: (t, h2)),
            pl.BlockSpec((_S, 512), lambda h2, t: (0, 2)),
        ],
        out_specs=pl.BlockSpec((_QT, 2 * _HD), lambda h2, t: (t, h2)),
        out_shape=jax.ShapeDtypeStruct((_S, _H * _HD), jnp.float32),
        scratch_shapes=[pltpu.VMEM((_S, _HD), jnp.bfloat16),
                        pltpu.VMEM((_S, _HD), jnp.bfloat16)],
    )(cos, sin, qln, kln, qkv, qkv)

    h, x2b, we = pl.pallas_call(
        _stage3_kernel,
        grid=(nt,),
        in_specs=[
            pl.BlockSpec((_ST, _D), lambda i: (i, 0)),
            pl.BlockSpec((_ST, _D), lambda i: (i, 0)),
            pl.BlockSpec((_D, _D), lambda i: (0, 0)),
            pl.BlockSpec((1, _D), lambda i: (0, 0)),
            pl.BlockSpec((_E, _D), lambda i: (0, 0)),
        ],
        out_specs=[
            pl.BlockSpec((_ST, _D), lambda i: (i, 0)),
            pl.BlockSpec((_ST, _D), lambda i: (i, 0)),
            pl.BlockSpec((_ST, _E), lambda i: (i, 0)),
        ],
        out_shape=[
            jax.ShapeDtypeStruct((_S, _D), jnp.float32),
            jax.ShapeDtypeStruct((_S, _D), jnp.bfloat16),
            jax.ShapeDtypeStruct((_S, _E), jnp.float32),
        ],
    )(x, ctx, wo_bf, ln2, Wr)

    ce = lambda c: (jnp.minimum(c, _E - 1), 0, 0)
    out = pl.pallas_call(
        _moe_kernel,
        grid=(_E + 1,),
        in_specs=[
            pl.BlockSpec((_S, _D), lambda c: (0, 0)),
            pl.BlockSpec((_S, _D), lambda c: (0, 0)),
            pl.BlockSpec((_S, _E), lambda c: (0, 0)),
            pl.BlockSpec((1, _DFF, _D), ce),
            pl.BlockSpec((1, _DFF, _D), ce),
            pl.BlockSpec((1, _D, _DFF), ce),
            pl.BlockSpec((_DFF, _D), lambda c: (0, 0)),
            pl.BlockSpec((_DFF, _D), lambda c: (0, 0)),
            pl.BlockSpec((_D, _DFF), lambda c: (0, 0)),
        ],
        out_specs=pl.BlockSpec((_S, _D), lambda c: (0, 0)),
        out_shape=jax.ShapeDtypeStruct((_S, _D), jnp.float32),
    )(x2b, h, we, Wg, Wu, Wd, Wsg, Wsu, Wsd)

    return out.reshape(1, _S, _D)


# speedup vs baseline: 3.3204x; 1.0118x over previous
import jax
import jax.numpy as jnp
from jax.experimental import pallas as pl
from jax.experimental.pallas import tpu as pltpu

_S, _D = 2048, 1024
_H, _HKV, _HD = 16, 4, 64
_E, _DFF = 8, 512
_G = _H // _HKV
_QKVD = (_H + 2 * _HKV) * _HD
_EPS = 1e-6
_THETA = 10000.0
_ST = 256
_QT = 1024


def _stage1_kernel(pos_ref, inv_ref, x_ref, w1_ref, wqkv_ref,
                   qkv_ref, cos_ref, sin_ref, wbf_ref):
    @pl.when(pl.program_id(0) == 0)
    def _():
        wbf_ref[...] = wqkv_ref[...].astype(jnp.bfloat16)

    x = x_ref[...]
    v = jnp.mean(x * x, axis=-1, keepdims=True)
    xn = w1_ref[...] * (x * jax.lax.rsqrt(v + _EPS))
    qkv_ref[...] = jax.lax.dot_general(
        xn.astype(jnp.bfloat16), wbf_ref[...],
        (((1,), (1,)), ((), ())), preferred_element_type=jnp.float32)
    f = pos_ref[...] * inv_ref[...]
    cos_ref[...] = jnp.cos(f)
    sin_ref[...] = jnp.sin(f)


def _rope(xn, c, s):
    x1 = xn[:, : _HD // 2]
    x2 = xn[:, _HD // 2:]
    return jnp.concatenate([x1 * c - x2 * s, x2 * c + x1 * s], axis=-1)


def _attn_kernel(cos_ref, sin_ref, qln_ref, kln_ref, q_ref, kv_ref,
                 ctx_ref, kr_ref, vb_ref):
    h2 = pl.program_id(0)
    t = pl.program_id(1)
    g2 = h2 // 2
    cq = cos_ref[pl.ds(t * _QT, _QT), :]
    sq = sin_ref[pl.ds(t * _QT, _QT), :]

    @pl.when(t == 0)
    def _():
        kv = kv_ref[...]
        ksel = jnp.where(
            g2 < 2,
            jnp.where(g2 == 0, kv[:, 0:_HD], kv[:, _HD:2 * _HD]),
            jnp.where(g2 == 2, kv[:, 2 * _HD:3 * _HD],
                      kv[:, 3 * _HD:4 * _HD]))
        vsel = jnp.where(
            g2 < 2,
            jnp.where(g2 == 0, kv[:, 4 * _HD:5 * _HD],
                      kv[:, 5 * _HD:6 * _HD]),
            jnp.where(g2 == 2, kv[:, 6 * _HD:7 * _HD],
                      kv[:, 7 * _HD:8 * _HD]))
        kvar = jnp.mean(ksel * ksel, axis=-1, keepdims=True)
        kn = kln_ref[...] * (ksel * jax.lax.rsqrt(kvar + _EPS))
        kr_ref[...] = _rope(kn, cos_ref[...], sin_ref[...]).astype(
            jnp.bfloat16)
        vb_ref[...] = jnp.concatenate(
            [vsel, jnp.ones((_S, 1), jnp.float32),
             jnp.zeros((_S, _HD - 1), jnp.float32)],
            axis=-1).astype(jnp.bfloat16)

    kr = kr_ref[...]
    vb = vb_ref[...]
    qpair = q_ref[...]
    qs = qln_ref[...] * 0.125

    def head(q):
        qv = jnp.mean(q * q, axis=-1, keepdims=True)
        qn = qs * (q * jax.lax.rsqrt(qv + _EPS))
        qr = _rope(qn, cq, sq).astype(jnp.bfloat16)
        scores = jax.lax.dot_general(
            qr, kr, (((1,), (1,)), ((), ())),
            preferred_element_type=jnp.float32)
        p = jnp.exp(scores.astype(jnp.bfloat16))
        ca = jax.lax.dot_general(
            p, vb, (((1,), (0,)), ((), ())),
            preferred_element_type=jnp.float32)
        return ca[:, :_HD] / ca[:, _HD:_HD + 1]

    ctx_ref[...] = jnp.concatenate(
        [head(qpair[:, :_HD]), head(qpair[:, _HD:])], axis=-1)


def _stage3_kernel(res_ref, ctx_ref, wo_ref, ln2_ref, wr_ref,
                   h_ref, x2b_ref, we_ref, wobf_ref):
    @pl.when(pl.program_id(0) == 0)
    def _():
        wobf_ref[...] = wo_ref[...].astype(jnp.bfloat16)

    ao = jax.lax.dot_general(
        ctx_ref[...].astype(jnp.bfloat16), wobf_ref[...],
        (((1,), (1,)), ((), ())), preferred_element_type=jnp.float32)
    h = res_ref[...] + ao
    h_ref[...] = h
    v = jnp.mean(h * h, axis=-1, keepdims=True)
    x2 = ln2_ref[...] * (h * jax.lax.rsqrt(v + _EPS))
    x2b_ref[...] = x2.astype(jnp.bfloat16)
    logits = jax.lax.dot_general(
        x2, wr_ref[...], (((1,), (1,)), ((), ())),
        preferred_element_type=jnp.float32)
    lm = jnp.max(logits, axis=-1, keepdims=True)
    el = jnp.exp(logits - lm)
    p = el / jnp.sum(el, axis=-1, keepdims=True)
    it = jax.lax.broadcasted_iota(jnp.int32, p.shape, 1)
    m1 = jnp.max(p, axis=-1, keepdims=True)
    i1 = jnp.min(jnp.where(p == m1, it, _E), axis=-1, keepdims=True)
    p2 = jnp.where(it == i1, -1.0, p)
    m2 = jnp.max(p2, axis=-1, keepdims=True)
    i2 = jnp.min(jnp.where(p2 == m2, it, _E), axis=-1, keepdims=True)
    we = jnp.where(it == i1, m1, 0.0) + jnp.where(it == i2, m2, 0.0)
    we_ref[...] = we / (m1 + m2)


def _moe_kernel(x2b_ref, h_ref, we_ref, wg_ref, wu_ref, wd_ref,
                wsg_ref, wsu_ref, wsd_ref, out_ref):
    c = pl.program_id(0)
    shared = c == _E
    x2b = x2b_ref[...]
    wg = jnp.where(shared, wsg_ref[...], wg_ref[0]).astype(jnp.bfloat16)
    wu = jnp.where(shared, wsu_ref[...], wu_ref[0]).astype(jnp.bfloat16)
    wd = jnp.where(shared, wsd_ref[...], wd_ref[0]).astype(jnp.bfloat16)
    g = jax.lax.dot_general(
        x2b, wg, (((1,), (1,)), ((), ())),
        preferred_element_type=jnp.float32)
    u = jax.lax.dot_general(
        x2b, wu, (((1,), (1,)), ((), ())),
        preferred_element_type=jnp.float32)
    a = (g * jax.nn.sigmoid(g)) * u
    it = jax.lax.broadcasted_iota(jnp.int32, we_ref.shape, 1)
    w = jnp.sum(jnp.where(it == jnp.minimum(c, _E - 1), we_ref[...], 0.0),
                axis=-1, keepdims=True)
    a = a * jnp.where(shared, 1.0, w)
    part = jax.lax.dot_general(
        a.astype(jnp.bfloat16), wd, (((1,), (1,)), ((), ())),
        preferred_element_type=jnp.float32)

    @pl.when(c == 0)
    def _():
        out_ref[...] = h_ref[...]

    out_ref[...] += part


def kernel(hidden_states, position_ids, ln1_w, Wqkv, q_ln_w, k_ln_w, Wo,
           ln2_w, Wr, Wg, Wu, Wd, Wsg, Wsu, Wsd):
    x = hidden_states.reshape(_S, _D)
    pos = position_ids.reshape(_S, 1).astype(jnp.float32)
    inv_freq = (1.0 / (_THETA ** (jnp.arange(0, _HD, 2, dtype=jnp.float32)
                                  / _HD))).reshape(1, _HD // 2)
    ln1 = ln1_w.reshape(1, _D)
    ln2 = ln2_w.reshape(1, _D)
    qln = q_ln_w.reshape(1, _HD)
    kln = k_ln_w.reshape(1, _HD)

    nt = _S // _ST
    qkv, cos, sin = pl.pallas_call(
        _stage1_kernel,
        grid=(nt,),
        in_specs=[
            pl.BlockSpec((_ST, 1), lambda i: (i, 0)),
            pl.BlockSpec((1, _HD // 2), lambda i: (0, 0)),
            pl.BlockSpec((_ST, _D), lambda i: (i, 0)),
            pl.BlockSpec((1, _D), lambda i: (0, 0)),
            pl.BlockSpec((_QKVD, _D), lambda i: (0, 0)),
        ],
        out_specs=[
            pl.BlockSpec((_ST, _QKVD), lambda i: (i, 0)),
            pl.BlockSpec((_ST, _HD // 2), lambda i: (i, 0)),
            pl.BlockSpec((_ST, _HD // 2), lambda i: (i, 0)),
        ],
        out_shape=[
            jax.ShapeDtypeStruct((_S, _QKVD), jnp.float32),
            jax.ShapeDtypeStruct((_S, _HD // 2), jnp.float32),
            jax.ShapeDtypeStruct((_S, _HD // 2), jnp.float32),
        ],
        scratch_shapes=[pltpu.VMEM((_QKVD, _D), jnp.bfloat16)],
    )(pos, inv_freq, x, ln1, Wqkv)

    ctx = pl.pallas_call(
        _attn_kernel,
        grid=(_H // 2, _S // _QT),
        in_specs=[
            pl.BlockSpec((_S, _HD // 2), lambda h2, t: (0, 0)),
            pl.BlockSpec((_S, _HD // 2), lambda h2, t: (0, 0)),
            pl.BlockSpec((1, _HD), lambda h2, t: (0, 0)),
            pl.BlockSpec((1, _HD), lambda h2, t: (0, 0)),
            pl.BlockSpec((_QT, 2 * _HD), lambda h2, t: (t, h2)),
            pl.BlockSpec((_S, 512), lambda h2, t: (0, 2)),
        ],
        out_specs=pl.BlockSpec((_QT, 2 * _HD), lambda h2, t: (t, h2)),
        out_shape=jax.ShapeDtypeStruct((_S, _H * _HD), jnp.float32),
        scratch_shapes=[pltpu.VMEM((_S, _HD), jnp.bfloat16),
                        pltpu.VMEM((_S, 2 * _HD), jnp.bfloat16)],
    )(cos, sin, qln, kln, qkv, qkv)

    h, x2b, we = pl.pallas_call(
        _stage3_kernel,
        grid=(nt,),
        in_specs=[
            pl.BlockSpec((_ST, _D), lambda i: (i, 0)),
            pl.BlockSpec((_ST, _D), lambda i: (i, 0)),
            pl.BlockSpec((_D, _D), lambda i: (0, 0)),
            pl.BlockSpec((1, _D), lambda i: (0, 0)),
            pl.BlockSpec((_E, _D), lambda i: (0, 0)),
        ],
        out_specs=[
            pl.BlockSpec((_ST, _D), lambda i: (i, 0)),
            pl.BlockSpec((_ST, _D), lambda i: (i, 0)),
            pl.BlockSpec((_ST, _E), lambda i: (i, 0)),
        ],
        out_shape=[
            jax.ShapeDtypeStruct((_S, _D), jnp.float32),
            jax.ShapeDtypeStruct((_S, _D), jnp.bfloat16),
            jax.ShapeDtypeStruct((_S, _E), jnp.float32),
        ],
        scratch_shapes=[pltpu.VMEM((_D, _D), jnp.bfloat16)],
    )(x, ctx, Wo, ln2, Wr)

    ce = lambda c: (jnp.minimum(c, _E - 1), 0, 0)
    out = pl.pallas_call(
        _moe_kernel,
        grid=(_E + 1,),
        in_specs=[
            pl.BlockSpec((_S, _D), lambda c: (0, 0)),
            pl.BlockSpec((_S, _D), lambda c: (0, 0)),
            pl.BlockSpec((_S, _E), lambda c: (0, 0)),
            pl.BlockSpec((1, _DFF, _D), ce),
            pl.BlockSpec((1, _DFF, _D), ce),
            pl.BlockSpec((1, _D, _DFF), ce),
            pl.BlockSpec((_DFF, _D), lambda c: (0, 0)),
            pl.BlockSpec((_DFF, _D), lambda c: (0, 0)),
            pl.BlockSpec((_D, _DFF), lambda c: (0, 0)),
        ],
        out_specs=pl.BlockSpec((_S, _D), lambda c: (0, 0)),
        out_shape=jax.ShapeDtypeStruct((_S, _D), jnp.float32),
    )(x2b, h, we, Wg, Wu, Wd, Wsg, Wsu, Wsd)

    return out.reshape(1, _S, _D)
